# baseline (speedup 1.0000x reference)
import dataclasses
import functools
import jax
import jax.numpy as jnp
from jax import lax
from jax.experimental import pallas as pl
from jax.experimental.pallas import tpu as pltpu
from jax.experimental.pallas import tpu_sc as plsc

_N = 10000
_E = 320000
_HID = 64
_EPS = 1e-5
_NEG = -3.0e38


def _node0_body(x_ref, wp_ref, bp_ref, wad_ref, was_ref, pd_ref, ps_ref):
    h = jnp.maximum(jnp.dot(x_ref[...], wp_ref[...],
                            preferred_element_type=jnp.float32) + bp_ref[...], 0.0)
    pd_ref[...] = jnp.dot(h, wad_ref[...], preferred_element_type=jnp.float32)
    ps_ref[...] = jnp.dot(h, was_ref[...], preferred_element_type=jnp.float32)


def _node0(x, Wp, bp, Wad, Was):
    out = [jax.ShapeDtypeStruct((_N, _HID), jnp.float32)] * 2
    return pl.pallas_call(
        _node0_body,
        out_shape=out,
    )(x, Wp, bp.reshape(1, -1), Wad, Was)


def _bn_next_body(agg_ref, g_ref, be_ref, wad_ref, was_ref, pd_ref, ps_ref):
    a = agg_ref[...]
    a = jnp.where(a > _NEG * 0.5, a, 0.0)
    mu = jnp.mean(a, axis=0, keepdims=True)
    var = jnp.mean((a - mu) * (a - mu), axis=0, keepdims=True)
    h = g_ref[...] * (a - mu) * jax.lax.rsqrt(var + _EPS) + be_ref[...]
    h = jnp.maximum(h, 0.0)
    pd_ref[...] = jnp.dot(h, wad_ref[...], preferred_element_type=jnp.float32)
    ps_ref[...] = jnp.dot(h, was_ref[...], preferred_element_type=jnp.float32)


def _bn_next(agg, g, be, Wad, Was):
    out = [jax.ShapeDtypeStruct((_N, _HID), jnp.float32)] * 2
    return pl.pallas_call(
        _bn_next_body,
        out_shape=out,
    )(agg, g.reshape(1, -1), be.reshape(1, -1), Wad, Was)


def _bn_final_body(agg_ref, g_ref, be_ref, wm1_ref, bm1_ref, wm2_ref, bm2_ref,
                   out_ref):
    a = agg_ref[...]
    a = jnp.where(a > _NEG * 0.5, a, 0.0)
    mu = jnp.mean(a, axis=0, keepdims=True)
    var = jnp.mean((a - mu) * (a - mu), axis=0, keepdims=True)
    h = g_ref[...] * (a - mu) * jax.lax.rsqrt(var + _EPS) + be_ref[...]
    h = jnp.maximum(h, 0.0)
    t = jnp.maximum(jnp.dot(h, wm1_ref[...],
                            preferred_element_type=jnp.float32) + bm1_ref[...], 0.0)
    out_ref[...] = jnp.dot(t, wm2_ref[...],
                           preferred_element_type=jnp.float32) + bm2_ref[...]


def _bn_final(agg, g, be, Wm1, bm1, Wm2, bm2):
    return pl.pallas_call(
        _bn_final_body,
        out_shape=jax.ShapeDtypeStruct((_N, Wm2.shape[1]), jnp.float32),
    )(agg, g.reshape(1, -1), be.reshape(1, -1), Wm1, bm1.reshape(1, -1),
      Wm2, bm2.reshape(1, -1))


def _edge_mlp_body(gd_ref, gs_ref, ea_ref, wae_ref, ba_ref, wb_ref, bb_ref,
                   m_ref):
    pre = gd_ref[...] + gs_ref[...] + jnp.dot(
        ea_ref[...], wae_ref[...],
        preferred_element_type=jnp.float32) + ba_ref[...]
    pre = jnp.maximum(pre, 0.0)
    m_ref[...] = jnp.dot(pre, wb_ref[...],
                         preferred_element_type=jnp.float32) + bb_ref[...]


def _edge_mlp(gd, gs, ea, Wae, ba, Wb, bb, block=8000):
    grid = _E // block
    return pl.pallas_call(
        _edge_mlp_body,
        grid=(grid,),
        in_specs=[
            pl.BlockSpec((block, _HID), lambda i: (i, 0)),
            pl.BlockSpec((block, _HID), lambda i: (i, 0)),
            pl.BlockSpec((block, ea.shape[1]), lambda i: (i, 0)),
            pl.BlockSpec(Wae.shape, lambda i: (0, 0)),
            pl.BlockSpec((1, _HID), lambda i: (0, 0)),
            pl.BlockSpec(Wb.shape, lambda i: (0, 0)),
            pl.BlockSpec((1, _HID), lambda i: (0, 0)),
        ],
        out_specs=pl.BlockSpec((block, _HID), lambda i: (i, 0)),
        out_shape=jax.ShapeDtypeStruct((_E, _HID), jnp.float32),
    )(gd, gs, ea, Wae, ba.reshape(1, -1), Wb, bb.reshape(1, -1))


_NC = 2
_NS = 16
_NW = _NC * _NS
_BPW = _E // _NW
_CH = 400
_NCHUNK = _BPW // _CH


def _sc_gather2(pd, ps, dst, src):
    mesh = plsc.VectorSubcoreMesh(core_axis_name="c", subcore_axis_name="s")

    @functools.partial(
        pl.kernel, mesh=mesh,
        compiler_params=pltpu.CompilerParams(use_tc_tiling_on_sc=False),
        out_type=[jax.ShapeDtypeStruct((_E, _HID), jnp.float32)] * 2,
        scratch_types=[
            pltpu.VMEM((_CH,), jnp.int32),
            pltpu.VMEM((_CH,), jnp.int32),
            pltpu.VMEM((_CH, _HID), jnp.float32),
            pltpu.VMEM((_CH, _HID), jnp.float32),
            pltpu.SemaphoreType.DMA,
            pltpu.SemaphoreType.DMA,
        ],
    )
    def k(pd_hbm, ps_hbm, dst_hbm, src_hbm, gd_hbm, gs_hbm,
          di_v, si_v, gd_v, gs_v, sem1, sem2):
        wid = lax.axis_index("s") * _NC + lax.axis_index("c")
        base = wid * _BPW

        @pl.loop(0, _NCHUNK)
        def _(j):
            off = base + j * _CH
            pltpu.sync_copy(dst_hbm.at[pl.ds(off, _CH)], di_v)
            pltpu.sync_copy(src_hbm.at[pl.ds(off, _CH)], si_v)
            a = pltpu.async_copy(pd_hbm.at[di_v], gd_v, sem1)
            b = pltpu.async_copy(ps_hbm.at[si_v], gs_v, sem2)
            a.wait()
            b.wait()
            pltpu.sync_copy(gd_v, gd_hbm.at[pl.ds(off, _CH)])
            pltpu.sync_copy(gs_v, gs_hbm.at[pl.ds(off, _CH)])

    return k(pd, ps, dst, src)


_NPAD = 10240
_RNG = _NPAD // _NW
_SB = 2000
_G = 64


def _sc_segmax(m, dst):
    mesh = plsc.VectorSubcoreMesh(core_axis_name="c", subcore_axis_name="s")
    nblk = _E // _SB
    cp = pltpu.CompilerParams(use_tc_tiling_on_sc=False)
    if "needs_layout_passes" in pltpu.CompilerParams.__dataclass_fields__:
        cp = dataclasses.replace(cp, needs_layout_passes=False)

    @functools.partial(
        pl.kernel, mesh=mesh,
        compiler_params=cp,
        out_type=jax.ShapeDtypeStruct((_NPAD, _HID), jnp.float32),
        scratch_types=[
            pltpu.VMEM((_SB,), jnp.int32),
            pltpu.VMEM((_SB,), jnp.int32),
            pltpu.VMEM((_SB,), jnp.int32),
            pltpu.VMEM((_G, _HID), jnp.float32),
            pltpu.VMEM((_G, _HID), jnp.float32),
            pltpu.VMEM((_RNG, _HID), jnp.float32),
            pltpu.VMEM((16,), jnp.int32),
            pltpu.SemaphoreType.DMA,
            pltpu.SemaphoreType.DMA,
        ],
    )
    def k(m_hbm, dst_hbm, agg_hbm, dblk_v, pid_v, pd_v, rows_a, rows_b,
          acc_v, off_r, sem_a, sem_b):
        wid = lax.axis_index("s") * _NC + lax.axis_index("c")
        lo = wid * _RNG
        hi = lo + _RNG
        lov = jnp.full((16,), lo, jnp.int32)
        hiv = jnp.full((16,), hi, jnp.int32)
        neg = jnp.full((16,), _NEG, jnp.float32)
        zero = jnp.zeros((16,), jnp.int32)
        iota = lax.iota(jnp.int32, 16)

        @pl.loop(0, _RNG)
        def _(i):
            for c in range(_HID // 16):
                plsc.store_scatter(acc_v, [jnp.full((16,), i, jnp.int32),
                                           iota + c * 16], neg)

        @pl.loop(0, _SB // 16)
        def _(i):
            pid_v.at[pl.ds(i * 16, 16)][...] = zero

        @pl.loop(0, nblk)
        def _(b):
            pltpu.sync_copy(dst_hbm.at[pl.ds(b * _SB, _SB)], dblk_v)

            off_r[...] = zero

            @pl.loop(0, _SB // 16, step=2)
            def _(i):
                off = off_r[...]
                for u in range(2):
                    d = dblk_v.at[pl.ds((i + u) * 16, 16)][...]
                    mask = (d >= lov) & (d < hiv)
                    mi = mask.astype(jnp.int32)
                    cs = plsc.cumsum(mi)
                    pos = off + cs - mi
                    ids = iota + jnp.full((16,), b * _SB, jnp.int32) + (i + u) * 16
                    plsc.store_scatter(pid_v, [pos], ids, mask=mask)
                    plsc.store_scatter(pd_v, [pos], d, mask=mask)
                    off = off + plsc.all_reduce_population_count(mask)
                off_r[...] = off

            cnt = jnp.max(off_r[...])
            nfull = (cnt + (_G - 1)) // _G
            ngrp_max = _SB // _G

            def accum(rows_v, g):
                rcount = jnp.minimum(cnt - g * _G, _G)

                def row_body(r, _):
                    dlv = plsc.load_gather(
                        pd_v, [jnp.full((16,), g * _G + r, jnp.int32)]) - lov
                    rv = jnp.full((16,), r, jnp.int32)
                    for c in range(_HID // 16):
                        colv = iota + c * 16
                        cur = plsc.load_gather(acc_v, [dlv, colv])
                        new = plsc.load_gather(rows_v, [rv, colv])
                        plsc.store_scatter(acc_v, [dlv, colv],
                                           jnp.maximum(cur, new))
                    return 0

                lax.fori_loop(0, rcount, row_body, 0)

            bufs = (rows_a, rows_b)
            sems = (sem_a, sem_b)
            copies = []
            for g in range(ngrp_max):
                cp_g = pltpu.make_async_copy(
                    m_hbm.at[pid_v.at[pl.ds(g * _G, _G)]],
                    bufs[g % 2], sems[g % 2])
                copies.append(cp_g)

            @pl.when(nfull > 0)
            def _():
                copies[0].start()

            for g in range(ngrp_max):
                @pl.when(jnp.int32(g) < nfull)
                def _(g=g):
                    if g + 1 < ngrp_max:
                        @pl.when(jnp.int32(g + 1) < nfull)
                        def _():
                            copies[g + 1].start()
                    copies[g].wait()
                    accum(bufs[g % 2], g)

        pltpu.sync_copy(acc_v, agg_hbm.at[pl.ds(lo, _RNG)])

    return k(m, dst)


def _segment_max(m, dst):
    return _sc_segmax(m, dst)[:_N]


def kernel(x, edge_index, edge_attr, Wp, bp, W0a, b0a, W0b, b0b, g0, be0,
           W1a, b1a, W1b, b1b, g1, be1, Wm1, bm1, Wm2, bm2):
    src = edge_index[0]
    dst = edge_index[1]
    H = _HID

    pd0, ps0 = _node0(x, Wp, bp, W0a[:H], W0a[H:2 * H])

    gd0, gs0 = _sc_gather2(pd0, ps0, dst, src)
    m0 = _edge_mlp(gd0, gs0, edge_attr, W0a[2 * H:], b0a, W0b, b0b)
    agg0 = _segment_max(m0, dst)

    pd1, ps1 = _bn_next(agg0, g0, be0, W1a[:H], W1a[H:2 * H])
    gd1, gs1 = _sc_gather2(pd1, ps1, dst, src)
    m1 = _edge_mlp(gd1, gs1, edge_attr, W1a[2 * H:], b1a, W1b, b1b)
    agg1 = _segment_max(m1, dst)

    return _bn_final(agg1, g1, be1, Wm1, bm1, Wm2, bm2)


# speedup vs baseline: 2.0761x; 2.0761x over previous
import dataclasses
import functools
import jax
import jax.numpy as jnp
from jax import lax
from jax.experimental import pallas as pl
from jax.experimental.pallas import tpu as pltpu
from jax.experimental.pallas import tpu_sc as plsc

_N = 10000
_E = 320000
_HID = 64
_EPS = 1e-5
_NEG = -3.0e38


def _node0_body(x_ref, wp_ref, bp_ref, wad_ref, was_ref, pd_ref, ps_ref):
    h = jnp.maximum(jnp.dot(x_ref[...], wp_ref[...],
                            preferred_element_type=jnp.float32) + bp_ref[...], 0.0)
    pd_ref[...] = jnp.dot(h, wad_ref[...], preferred_element_type=jnp.float32)
    ps_ref[...] = jnp.dot(h, was_ref[...], preferred_element_type=jnp.float32)


def _node0(x, Wp, bp, Wad, Was):
    out = [jax.ShapeDtypeStruct((_N, _HID), jnp.float32)] * 2
    return pl.pallas_call(
        _node0_body,
        out_shape=out,
    )(x, Wp, bp.reshape(1, -1), Wad, Was)


def _bn_next_body(agg_ref, g_ref, be_ref, wad_ref, was_ref, pd_ref, ps_ref):
    a = agg_ref[...]
    a = jnp.where(a > _NEG * 0.5, a, 0.0)
    mu = jnp.mean(a, axis=0, keepdims=True)
    var = jnp.mean((a - mu) * (a - mu), axis=0, keepdims=True)
    h = g_ref[...] * (a - mu) * jax.lax.rsqrt(var + _EPS) + be_ref[...]
    h = jnp.maximum(h, 0.0)
    pd_ref[...] = jnp.dot(h, wad_ref[...], preferred_element_type=jnp.float32)
    ps_ref[...] = jnp.dot(h, was_ref[...], preferred_element_type=jnp.float32)


def _bn_next(agg, g, be, Wad, Was):
    out = [jax.ShapeDtypeStruct((_N, _HID), jnp.float32)] * 2
    return pl.pallas_call(
        _bn_next_body,
        out_shape=out,
    )(agg, g.reshape(1, -1), be.reshape(1, -1), Wad, Was)


def _bn_final_body(agg_ref, g_ref, be_ref, wm1_ref, bm1_ref, wm2_ref, bm2_ref,
                   out_ref):
    a = agg_ref[...]
    a = jnp.where(a > _NEG * 0.5, a, 0.0)
    mu = jnp.mean(a, axis=0, keepdims=True)
    var = jnp.mean((a - mu) * (a - mu), axis=0, keepdims=True)
    h = g_ref[...] * (a - mu) * jax.lax.rsqrt(var + _EPS) + be_ref[...]
    h = jnp.maximum(h, 0.0)
    t = jnp.maximum(jnp.dot(h, wm1_ref[...],
                            preferred_element_type=jnp.float32) + bm1_ref[...], 0.0)
    out_ref[...] = jnp.dot(t, wm2_ref[...],
                           preferred_element_type=jnp.float32) + bm2_ref[...]


def _bn_final(agg, g, be, Wm1, bm1, Wm2, bm2):
    return pl.pallas_call(
        _bn_final_body,
        out_shape=jax.ShapeDtypeStruct((_N, Wm2.shape[1]), jnp.float32),
    )(agg, g.reshape(1, -1), be.reshape(1, -1), Wm1, bm1.reshape(1, -1),
      Wm2, bm2.reshape(1, -1))


def _edge_mlp_body(gd_ref, gs_ref, ea_ref, wae_ref, ba_ref, wb_ref, bb_ref,
                   m_ref):
    pre = gd_ref[...] + gs_ref[...] + jnp.dot(
        ea_ref[...], wae_ref[...],
        preferred_element_type=jnp.float32) + ba_ref[...]
    pre = jnp.maximum(pre, 0.0)
    m_ref[...] = jnp.dot(pre, wb_ref[...],
                         preferred_element_type=jnp.float32) + bb_ref[...]


def _edge_mlp(gd, gs, ea, Wae, ba, Wb, bb, block=8000):
    grid = _E // block
    return pl.pallas_call(
        _edge_mlp_body,
        grid=(grid,),
        in_specs=[
            pl.BlockSpec((block, _HID), lambda i: (i, 0)),
            pl.BlockSpec((block, _HID), lambda i: (i, 0)),
            pl.BlockSpec((block, ea.shape[1]), lambda i: (i, 0)),
            pl.BlockSpec(Wae.shape, lambda i: (0, 0)),
            pl.BlockSpec((1, _HID), lambda i: (0, 0)),
            pl.BlockSpec(Wb.shape, lambda i: (0, 0)),
            pl.BlockSpec((1, _HID), lambda i: (0, 0)),
        ],
        out_specs=pl.BlockSpec((block, _HID), lambda i: (i, 0)),
        out_shape=jax.ShapeDtypeStruct((_E, _HID), jnp.float32),
    )(gd, gs, ea, Wae, ba.reshape(1, -1), Wb, bb.reshape(1, -1))


_NC = 2
_NS = 16
_NW = _NC * _NS
_BPW = _E // _NW
_CH = 400
_NCHUNK = _BPW // _CH


def _sc_gather2(pd, ps, dst, src):
    mesh = plsc.VectorSubcoreMesh(core_axis_name="c", subcore_axis_name="s")

    @functools.partial(
        pl.kernel, mesh=mesh,
        compiler_params=pltpu.CompilerParams(use_tc_tiling_on_sc=False),
        out_type=[jax.ShapeDtypeStruct((_E, _HID), jnp.float32)] * 2,
        scratch_types=[
            pltpu.VMEM((_CH,), jnp.int32),
            pltpu.VMEM((_CH,), jnp.int32),
            pltpu.VMEM((_CH, _HID), jnp.float32),
            pltpu.VMEM((_CH, _HID), jnp.float32),
            pltpu.SemaphoreType.DMA,
            pltpu.SemaphoreType.DMA,
        ],
    )
    def k(pd_hbm, ps_hbm, dst_hbm, src_hbm, gd_hbm, gs_hbm,
          di_v, si_v, gd_v, gs_v, sem1, sem2):
        wid = lax.axis_index("s") * _NC + lax.axis_index("c")
        base = wid * _BPW

        @pl.loop(0, _NCHUNK)
        def _(j):
            off = base + j * _CH
            pltpu.sync_copy(dst_hbm.at[pl.ds(off, _CH)], di_v)
            pltpu.sync_copy(src_hbm.at[pl.ds(off, _CH)], si_v)
            a = pltpu.async_copy(pd_hbm.at[di_v], gd_v, sem1)
            b = pltpu.async_copy(ps_hbm.at[si_v], gs_v, sem2)
            a.wait()
            b.wait()
            pltpu.sync_copy(gd_v, gd_hbm.at[pl.ds(off, _CH)])
            pltpu.sync_copy(gs_v, gs_hbm.at[pl.ds(off, _CH)])

    return k(pd, ps, dst, src)


_NPAD = 10240
_RNG = _NPAD // _NW
_SB = 2000
_G = 64


def _sc_segmax(m, dst):
    mesh = plsc.VectorSubcoreMesh(core_axis_name="c", subcore_axis_name="s")
    nblk = _E // _SB
    cp = pltpu.CompilerParams(use_tc_tiling_on_sc=False)
    if "needs_layout_passes" in pltpu.CompilerParams.__dataclass_fields__:
        cp = dataclasses.replace(cp, needs_layout_passes=False)

    @functools.partial(
        pl.kernel, mesh=mesh,
        compiler_params=cp,
        out_type=jax.ShapeDtypeStruct((_NPAD, _HID), jnp.float32),
        scratch_types=[
            pltpu.VMEM((_SB,), jnp.int32),
            pltpu.VMEM((_SB,), jnp.int32),
            pltpu.VMEM((_SB,), jnp.int32),
            pltpu.VMEM((_G, _HID), jnp.float32),
            pltpu.VMEM((_G, _HID), jnp.float32),
            pltpu.VMEM((_RNG, _HID), jnp.float32),
            pltpu.VMEM((16,), jnp.int32),
            pltpu.SemaphoreType.DMA,
            pltpu.SemaphoreType.DMA,
        ],
    )
    def k(m_hbm, dst_hbm, agg_hbm, dblk_v, pid_v, pd_v, rows_a, rows_b,
          acc_v, off_r, sem_a, sem_b):
        wid = lax.axis_index("s") * _NC + lax.axis_index("c")
        lo = wid * _RNG
        hi = lo + _RNG
        lov = jnp.full((16,), lo, jnp.int32)
        hiv = jnp.full((16,), hi, jnp.int32)
        neg = jnp.full((16,), _NEG, jnp.float32)
        zero = jnp.zeros((16,), jnp.int32)
        iota = lax.iota(jnp.int32, 16)

        @pl.loop(0, _RNG)
        def _(i):
            for c in range(_HID // 16):
                plsc.store_scatter(acc_v, [jnp.full((16,), i, jnp.int32),
                                           iota + c * 16], neg)

        @pl.loop(0, _SB // 16)
        def _(i):
            pid_v.at[pl.ds(i * 16, 16)][...] = zero

        @pl.loop(0, nblk)
        def _(b):
            pltpu.sync_copy(dst_hbm.at[pl.ds(b * _SB, _SB)], dblk_v)

            def scan_body(i, off):
                d = dblk_v.at[pl.ds(i * 16, 16)][...]
                mask = (d >= lov) & (d < hiv)
                mi = mask.astype(jnp.int32)
                cs = plsc.cumsum(mi)
                pos = off + cs - mi
                ids = iota + jnp.full((16,), b * _SB + i * 16, jnp.int32)
                plsc.store_scatter(pid_v, [pos], ids, mask=mask)
                plsc.store_scatter(pd_v, [pos], d, mask=mask)
                return off + plsc.all_reduce_population_count(mask)

            off = lax.fori_loop(0, _SB // 16, scan_body, zero)
            cnt = jnp.max(off)
            nfull = (cnt + (_G - 1)) // _G
            ngrp_max = _SB // _G

            def accum(rows_v, g):
                rcount = jnp.minimum(cnt - g * _G, _G)

                def row_body(r, _):
                    dlv = plsc.load_gather(
                        pd_v, [jnp.full((16,), g * _G + r, jnp.int32)]) - lov
                    rv = jnp.full((16,), r, jnp.int32)
                    for c in range(_HID // 16):
                        colv = iota + c * 16
                        cur = plsc.load_gather(acc_v, [dlv, colv])
                        new = plsc.load_gather(rows_v, [rv, colv])
                        plsc.store_scatter(acc_v, [dlv, colv],
                                           jnp.maximum(cur, new))
                    return 0

                lax.fori_loop(0, rcount, row_body, 0)

            bufs = (rows_a, rows_b)
            sems = (sem_a, sem_b)
            copies = []
            for g in range(ngrp_max):
                cp_g = pltpu.make_async_copy(
                    m_hbm.at[pid_v.at[pl.ds(g * _G, _G)]],
                    bufs[g % 2], sems[g % 2])
                copies.append(cp_g)

            @pl.when(nfull > 0)
            def _():
                copies[0].start()

            for g in range(ngrp_max):
                @pl.when(jnp.int32(g) < nfull)
                def _(g=g):
                    if g + 1 < ngrp_max:
                        @pl.when(jnp.int32(g + 1) < nfull)
                        def _():
                            copies[g + 1].start()
                    copies[g].wait()
                    accum(bufs[g % 2], g)

        pltpu.sync_copy(acc_v, agg_hbm.at[pl.ds(lo, _RNG)])

    return k(m, dst)


def _segment_max(m, dst):
    return jax.ops.segment_max(m, dst, num_segments=_N)


def kernel(x, edge_index, edge_attr, Wp, bp, W0a, b0a, W0b, b0b, g0, be0,
           W1a, b1a, W1b, b1b, g1, be1, Wm1, bm1, Wm2, bm2):
    src = edge_index[0]
    dst = edge_index[1]
    H = _HID

    pd0, ps0 = _node0(x, Wp, bp, W0a[:H], W0a[H:2 * H])

    gd0, gs0 = _sc_gather2(pd0, ps0, dst, src)
    m0 = _edge_mlp(gd0, gs0, edge_attr, W0a[2 * H:], b0a, W0b, b0b)
    agg0 = _segment_max(m0, dst)

    pd1, ps1 = _bn_next(agg0, g0, be0, W1a[:H], W1a[H:2 * H])
    gd1, gs1 = _sc_gather2(pd1, ps1, dst, src)
    m1 = _edge_mlp(gd1, gs1, edge_attr, W1a[2 * H:], b1a, W1b, b1b)
    agg1 = _segment_max(m1, dst)

    return _bn_final(agg1, g1, be1, Wm1, bm1, Wm2, bm2)
